# Initial kernel scaffold; baseline (speedup 1.0000x reference)
import jax
import jax.numpy as jnp
from jax.experimental import pallas as pl


def kernel(x, y, y_atomtypes, W1, b1, W2, b2, W3, b3, bn1_g, bn1_b, bn2_g, bn2_b, x_batch, y_batch):
    raise NotImplementedError("write your pallas kernel here")


# speedup vs baseline: 8.1215x; 8.1215x over previous
import functools

import jax
import jax.numpy as jnp
import numpy as np
from jax import lax
from jax.experimental import pallas as pl
from jax.experimental.pallas import tpu as pltpu
from jax.experimental.pallas import tpu_sc as plsc

N = 8192
M = 8192
D = 16
K = 16

BX = 256
CY = 1024
NX = N // BX
NCY = M // CY

_INF = 1e30
_BIGI = 1e9


def _topk_body(xblo, xbhi, yblo, ybhi, xa_ref, yt_ref, xbf_ref, ybf_ref,
               oidx_ref, topv, topi):
    i = pl.program_id(0)
    j = pl.program_id(1)

    @pl.when(j == 0)
    def _init():
        topv[...] = jnp.full((BX, K), _INF, dtype=jnp.float32)
        topi[...] = jnp.zeros((BX, K), dtype=jnp.float32)

    overlap = (xblo[i] <= ybhi[j]) & (yblo[j] <= xbhi[i])

    @pl.when(overlap)
    def _merge():
        xa = xa_ref[...]
        yt = yt_ref[...]
        dot = lax.dot_general(xa, yt, (((1,), (0,)), ((), ())),
                              preferred_element_type=jnp.float32)
        sx = jnp.sum(xa * xa, axis=1, keepdims=True)
        sy = jnp.sum(yt * yt, axis=0, keepdims=True)
        d2 = (sx + sy) - 2.0 * dot
        pen = jnp.where(xbf_ref[...] != ybf_ref[...], 1e10, 0.0).astype(jnp.float32)
        d2 = d2 + pen

        colbase = (j * CY).astype(jnp.float32)
        colid = lax.broadcasted_iota(jnp.int32, (BX, CY), 1).astype(jnp.float32) + colbase

        A = jnp.concatenate([d2, topv[...]], axis=1)
        I = jnp.concatenate([colid, topi[...]], axis=1)

        def rnd(r, c):
            a, ev, ei = c
            m = jnp.min(a, axis=1, keepdims=True)
            ii = jnp.min(jnp.where(a <= m, I, _BIGI), axis=1, keepdims=True)
            a = jnp.where(I == ii, _INF, a)
            lane = lax.broadcasted_iota(jnp.int32, (BX, K), 1)
            ev = jnp.where(lane == r, m, ev)
            ei = jnp.where(lane == r, ii, ei)
            return (a, ev, ei)

        _, ev, ei = lax.fori_loop(
            0, K, rnd,
            (A, jnp.zeros((BX, K), jnp.float32), jnp.zeros((BX, K), jnp.float32)))
        topv[...] = ev
        topi[...] = ei

    @pl.when(j == NCY - 1)
    def _out():
        oidx_ref[...] = topi[...].astype(jnp.int32)


def _run_topk(xa, yt, xbf, ybf, xblo, xbhi, yblo, ybhi):
    grid_spec = pltpu.PrefetchScalarGridSpec(
        num_scalar_prefetch=4,
        grid=(NX, NCY),
        in_specs=[
            pl.BlockSpec((BX, 8), lambda i, j, *_: (i, 0)),
            pl.BlockSpec((8, CY), lambda i, j, *_: (0, j)),
            pl.BlockSpec((BX, 1), lambda i, j, *_: (i, 0)),
            pl.BlockSpec((1, CY), lambda i, j, *_: (0, j)),
        ],
        out_specs=pl.BlockSpec((BX, K), lambda i, j, *_: (i, 0)),
        scratch_shapes=[
            pltpu.VMEM((BX, K), jnp.float32),
            pltpu.VMEM((BX, K), jnp.float32),
        ],
    )
    return pl.pallas_call(
        _topk_body,
        grid_spec=grid_spec,
        out_shape=jax.ShapeDtypeStruct((N, K), jnp.int32),
    )(xblo, xbhi, yblo, ybhi, xa, yt, xbf, ybf)


_NW = 32
_BPW = (N * K) // _NW
_HALF = _BPW // 2
_GCH = 128


def _sc_gather_body(tab_hbm, idx_hbm, out_hbm, idx_v, rows_v, sem):
    wid = lax.axis_index("s") * 2 + lax.axis_index("c")
    base = wid * _BPW
    pltpu.sync_copy(idx_hbm.at[pl.ds(base, _BPW)], idx_v)
    for h in range(2):
        hoff = h * _HALF
        copies = []
        for c in range(_HALF // _GCH):
            src = tab_hbm.at[idx_v.at[pl.ds(hoff + c * _GCH, _GCH)]]
            dst = rows_v.at[pl.ds(c * _GCH, _GCH)]
            copies.append(pltpu.async_copy(src, dst, sem))
        for cp in copies:
            cp.wait()
        pltpu.sync_copy(rows_v, out_hbm.at[pl.ds(base + hoff, _HALF)])


def _run_gather(tab, idxf):
    mesh = plsc.VectorSubcoreMesh(core_axis_name="c", subcore_axis_name="s")
    kfn = functools.partial(
        pl.kernel,
        mesh=mesh,
        compiler_params=pltpu.CompilerParams(use_tc_tiling_on_sc=False),
        out_type=jax.ShapeDtypeStruct((N * K, 32), jnp.float32),
        scratch_types=[
            pltpu.VMEM((_BPW,), jnp.int32),
            pltpu.VMEM((_HALF, 32), jnp.float32),
            pltpu.SemaphoreType.DMA,
        ],
    )(_sc_gather_body)
    return kfn(tab, idxf)


def _mlp_body(g_ref, xe_ref, w1_ref, w2_ref, w3_ref, prm_ref, out_ref):
    g = g_ref[...]
    xe = xe_ref[...]
    lane = lax.broadcasted_iota(jnp.int32, (1, 1, 32), 2)
    cmask = ((lane >= D) & (lane < D + 3)).astype(jnp.float32)
    amask = (lane < D).astype(jnp.float32)
    dhot = (lane == D).astype(jnp.float32)

    diff = (g - xe[:, None, :]) * cmask
    dists = jnp.sum(diff * diff, axis=2)
    dinv = 1.0 / dists

    feat = g * amask + dinv[:, :, None] * dhot
    feat2 = feat.reshape(BX * K, 32)

    s = float(np.sqrt(1.0 + 1e-5))
    b1 = prm_ref[0:1, :]
    g1 = prm_ref[1:2, :]
    be1 = prm_ref[2:3, :]
    b2 = prm_ref[3:4, :]
    g2 = prm_ref[4:5, :]
    be2 = prm_ref[5:6, :]
    b3 = prm_ref[6:7, :]

    fx = lax.dot_general(feat2, w1_ref[...], (((1,), (0,)), ((), ())),
                         preferred_element_type=jnp.float32) + b1
    fx = jnp.where(fx >= 0, fx, 0.2 * fx)
    fx = fx / s * g1 + be1
    fx1 = jnp.sum(fx.reshape(BX, K, D), axis=1)

    fy = lax.dot_general(fx, w2_ref[...], (((1,), (0,)), ((), ())),
                         preferred_element_type=jnp.float32) + b2
    fy = jnp.where(fy >= 0, fy, 0.2 * fy)
    fy = fy / s * g2 + be2
    fx2 = jnp.sum(fy.reshape(BX, K, D), axis=1)

    cc = jnp.concatenate([fx1, fx2], axis=1)
    out_ref[...] = lax.dot_general(cc, w3_ref[...], (((1,), (0,)), ((), ())),
                                   preferred_element_type=jnp.float32) + b3


def _run_mlp(g3, xe, w1p, w2p, w3p, prm):
    return pl.pallas_call(
        _mlp_body,
        grid=(NX,),
        in_specs=[
            pl.BlockSpec((BX, K, 32), lambda i: (i, 0, 0)),
            pl.BlockSpec((BX, 32), lambda i: (i, 0)),
            pl.BlockSpec((32, D), lambda i: (0, 0)),
            pl.BlockSpec((D, D), lambda i: (0, 0)),
            pl.BlockSpec((2 * D, D), lambda i: (0, 0)),
            pl.BlockSpec((8, D), lambda i: (0, 0)),
        ],
        out_specs=pl.BlockSpec((BX, D), lambda i: (i, 0)),
        out_shape=jax.ShapeDtypeStruct((N, D), jnp.float32),
    )(g3, xe, w1p, w2p, w3p, prm)


def kernel(x, y, y_atomtypes, W1, b1, W2, b2, W3, b3,
           bn1_g, bn1_b, bn2_g, bn2_b, x_batch, y_batch):
    f32 = jnp.float32
    x = x.astype(f32)
    y = y.astype(f32)

    xa = jnp.concatenate([x, jnp.zeros((N, 5), f32)], axis=1)
    yt = jnp.concatenate([y, jnp.zeros((M, 5), f32)], axis=1).T
    xbf = x_batch.astype(f32)[:, None]
    ybf = y_batch.astype(f32)[None, :]
    xb2 = x_batch.reshape(NX, BX).astype(jnp.int32)
    yb2 = y_batch.reshape(NCY, CY).astype(jnp.int32)
    xblo, xbhi = xb2.min(axis=1), xb2.max(axis=1)
    yblo, ybhi = yb2.min(axis=1), yb2.max(axis=1)

    idx = _run_topk(xa, yt, xbf, ybf, xblo, xbhi, yblo, ybhi)

    tab = jnp.concatenate([y_atomtypes.astype(f32), y,
                           jnp.zeros((M, 13), f32)], axis=1)
    gath = _run_gather(tab, idx.reshape(N * K))

    g3 = gath.reshape(N, K, 32)
    xe = jnp.concatenate([jnp.zeros((N, D), f32), x,
                          jnp.zeros((N, 32 - D - 3), f32)], axis=1)
    w1p = jnp.concatenate([W1[:, :D].T, W1[:, D:D + 1].T,
                           jnp.zeros((32 - D - 1, D), f32)], axis=0)
    w2p = W2.T
    w3p = W3.T
    prm = jnp.stack([b1, bn1_g, bn1_b, b2, bn2_g, bn2_b, b3,
                     jnp.zeros((D,), f32)], axis=0)

    return _run_mlp(g3, xe, w1p, w2p, w3p, prm)


# speedup vs baseline: 8.6862x; 1.0695x over previous
import functools

import jax
import jax.numpy as jnp
import numpy as np
from jax import lax
from jax.experimental import pallas as pl
from jax.experimental.pallas import tpu as pltpu
from jax.experimental.pallas import tpu_sc as plsc

N = 8192
M = 8192
D = 16
K = 16

BX = 256
CY = 1024
NX = N // BX
NCY = M // CY

_INF = 1e30
_BIGI = 1e9


def _topk_body(xblo, xbhi, yblo, ybhi, xa_ref, yt_ref, xbf_ref, ybf_ref,
               oidx_ref, topv, topi):
    i = pl.program_id(0)
    j = pl.program_id(1)

    @pl.when(j == 0)
    def _init():
        topv[...] = jnp.full((BX, K), _INF, dtype=jnp.float32)
        topi[...] = jnp.zeros((BX, K), dtype=jnp.float32)

    overlap = (xblo[i] <= ybhi[j]) & (yblo[j] <= xbhi[i])

    @pl.when(overlap)
    def _merge():
        xa = xa_ref[...]
        yt = yt_ref[...]
        dot = lax.dot_general(xa, yt, (((1,), (0,)), ((), ())),
                              preferred_element_type=jnp.float32)
        sx = jnp.sum(xa * xa, axis=1, keepdims=True)
        sy = jnp.sum(yt * yt, axis=0, keepdims=True)
        d2 = (sx + sy) - 2.0 * dot
        pen = jnp.where(xbf_ref[...] != ybf_ref[...], 1e10, 0.0).astype(jnp.float32)
        d2 = d2 + pen

        colbase = (j * CY).astype(jnp.float32)
        colid = lax.broadcasted_iota(jnp.int32, (BX, CY), 1).astype(jnp.float32) + colbase

        A = jnp.concatenate([d2, topv[...]], axis=1)
        I = jnp.concatenate([colid, topi[...]], axis=1)

        def rnd(r, c):
            a, ev, ei = c
            m = jnp.min(a, axis=1, keepdims=True)
            ii = jnp.min(jnp.where(a <= m, I, _BIGI), axis=1, keepdims=True)
            a = jnp.where(I == ii, _INF, a)
            lane = lax.broadcasted_iota(jnp.int32, (BX, K), 1)
            ev = jnp.where(lane == r, m, ev)
            ei = jnp.where(lane == r, ii, ei)
            return (a, ev, ei)

        _, ev, ei = lax.fori_loop(
            0, K, rnd,
            (A, jnp.zeros((BX, K), jnp.float32), jnp.zeros((BX, K), jnp.float32)))
        topv[...] = ev
        topi[...] = ei

    @pl.when(j == NCY - 1)
    def _out():
        oidx_ref[...] = topi[...].astype(jnp.int32)


def _run_topk(xa, yt, xbf, ybf, xblo, xbhi, yblo, ybhi):
    grid_spec = pltpu.PrefetchScalarGridSpec(
        num_scalar_prefetch=4,
        grid=(NX, NCY),
        in_specs=[
            pl.BlockSpec((BX, 8), lambda i, j, *_: (i, 0)),
            pl.BlockSpec((8, CY), lambda i, j, *_: (0, j)),
            pl.BlockSpec((BX, 1), lambda i, j, *_: (i, 0)),
            pl.BlockSpec((1, CY), lambda i, j, *_: (0, j)),
        ],
        out_specs=pl.BlockSpec((BX, K), lambda i, j, *_: (i, 0)),
        scratch_shapes=[
            pltpu.VMEM((BX, K), jnp.float32),
            pltpu.VMEM((BX, K), jnp.float32),
        ],
    )
    return pl.pallas_call(
        _topk_body,
        grid_spec=grid_spec,
        out_shape=jax.ShapeDtypeStruct((N, K), jnp.int32),
    )(xblo, xbhi, yblo, ybhi, xa, yt, xbf, ybf)


_NW = 32
_BPW = (N * K) // _NW
_HALF = _BPW // 2
_GCH = 128


def _sc_gather_body(tab_hbm, idx_hbm, out_hbm, idx_v, rows_v, sem):
    wid = lax.axis_index("s") * 2 + lax.axis_index("c")
    base = wid * _BPW
    pltpu.sync_copy(idx_hbm.at[pl.ds(base, _BPW)], idx_v)
    for h in range(2):
        hoff = h * _HALF
        copies = []
        for c in range(_HALF // _GCH):
            src = tab_hbm.at[idx_v.at[pl.ds(hoff + c * _GCH, _GCH)]]
            dst = rows_v.at[pl.ds(c * _GCH, _GCH)]
            copies.append(pltpu.async_copy(src, dst, sem))
        for cp in copies:
            cp.wait()
        pltpu.sync_copy(rows_v, out_hbm.at[pl.ds(base + hoff, _HALF)])


def _run_gather(tab, idxf):
    mesh = plsc.VectorSubcoreMesh(core_axis_name="c", subcore_axis_name="s")
    kfn = functools.partial(
        pl.kernel,
        mesh=mesh,
        compiler_params=pltpu.CompilerParams(use_tc_tiling_on_sc=False),
        out_type=jax.ShapeDtypeStruct((N * K, 32), jnp.float32),
        scratch_types=[
            pltpu.VMEM((_BPW,), jnp.int32),
            pltpu.VMEM((_HALF, 32), jnp.float32),
            pltpu.SemaphoreType.DMA,
        ],
    )(_sc_gather_body)
    return kfn(tab, idxf)


_KW = K * 32
_KD = K * D


def _mlp_body(g_ref, xe_ref, w1_ref, w2_ref, w3_ref, prm_ref, out_ref):
    gl = g_ref[...]
    xe = xe_ref[...]
    xew = jnp.concatenate([xe] * K, axis=1)

    lane = lax.broadcasted_iota(jnp.int32, (1, _KW), 1)
    lm = lax.rem(lane, 32)
    cmaskl = ((lm >= D) & (lm < D + 3)).astype(jnp.float32)
    amaskl = (lm < D).astype(jnp.float32)
    dhotl = lm == D

    diff = (gl - xew) * cmaskl
    sq = diff * diff
    t = (sq + jnp.roll(sq, -1, axis=1)) + jnp.roll(sq, -2, axis=1)
    rec = 1.0 / t
    featl = gl * amaskl + jnp.where(dhotl, rec, 0.0)

    s = float(np.sqrt(1.0 + 1e-5))
    b1 = prm_ref[0:1, :]
    g1 = prm_ref[1:2, :]
    be1 = prm_ref[2:3, :]
    b2 = prm_ref[3:4, :]
    g2 = prm_ref[4:5, :]
    be2 = prm_ref[5:6, :]
    b3 = prm_ref[6:7, 0:D]

    fx = lax.dot_general(featl, w1_ref[...], (((1,), (0,)), ((), ())),
                         preferred_element_type=jnp.float32) + b1
    fx = jnp.where(fx >= 0, fx, 0.2 * fx)
    fx = fx / s * g1 + be1

    fy = lax.dot_general(fx, w2_ref[...], (((1,), (0,)), ((), ())),
                         preferred_element_type=jnp.float32) + b2
    fy = jnp.where(fy >= 0, fy, 0.2 * fy)
    fy = fy / s * g2 + be2

    cc = jnp.concatenate([fx, fy], axis=1)
    out_ref[...] = lax.dot_general(cc, w3_ref[...], (((1,), (0,)), ((), ())),
                                   preferred_element_type=jnp.float32) + b3


def _run_mlp(g3, xe, w1p, w2p, w3p, prm):
    return pl.pallas_call(
        _mlp_body,
        grid=(NX,),
        in_specs=[
            pl.BlockSpec((BX, _KW), lambda i: (i, 0)),
            pl.BlockSpec((BX, 32), lambda i: (i, 0)),
            pl.BlockSpec((_KW, _KD), lambda i: (0, 0)),
            pl.BlockSpec((_KD, _KD), lambda i: (0, 0)),
            pl.BlockSpec((2 * _KD, D), lambda i: (0, 0)),
            pl.BlockSpec((8, _KD), lambda i: (0, 0)),
        ],
        out_specs=pl.BlockSpec((BX, D), lambda i: (i, 0)),
        out_shape=jax.ShapeDtypeStruct((N, D), jnp.float32),
    )(g3, xe, w1p, w2p, w3p, prm)


def kernel(x, y, y_atomtypes, W1, b1, W2, b2, W3, b3,
           bn1_g, bn1_b, bn2_g, bn2_b, x_batch, y_batch):
    f32 = jnp.float32
    x = x.astype(f32)
    y = y.astype(f32)

    xa = jnp.concatenate([x, jnp.zeros((N, 5), f32)], axis=1)
    yt = jnp.concatenate([y, jnp.zeros((M, 5), f32)], axis=1).T
    xbf = x_batch.astype(f32)[:, None]
    ybf = y_batch.astype(f32)[None, :]
    xb2 = x_batch.reshape(NX, BX).astype(jnp.int32)
    yb2 = y_batch.reshape(NCY, CY).astype(jnp.int32)
    xblo, xbhi = xb2.min(axis=1), xb2.max(axis=1)
    yblo, ybhi = yb2.min(axis=1), yb2.max(axis=1)

    idx = _run_topk(xa, yt, xbf, ybf, xblo, xbhi, yblo, ybhi)

    tab = jnp.concatenate([y_atomtypes.astype(f32), y,
                           jnp.zeros((M, 13), f32)], axis=1)
    gath = _run_gather(tab, idx.reshape(N * K))

    gl = gath.reshape(N, K * 32)
    xe = jnp.concatenate([jnp.zeros((N, D), f32), x,
                          jnp.zeros((N, 32 - D - 3), f32)], axis=1)
    w1p = jnp.concatenate([W1[:, :D].T, W1[:, D:D + 1].T,
                           jnp.zeros((32 - D - 1, D), f32)], axis=0)
    eyek = jnp.eye(K, dtype=f32)
    w1bd = jnp.kron(eyek, w1p)
    w2bd = jnp.kron(eyek, W2.T)
    w3cat = jnp.concatenate([jnp.tile(W3.T[:D], (K, 1)),
                             jnp.tile(W3.T[D:], (K, 1))], axis=0)
    b3p = jnp.concatenate([b3, jnp.zeros((K * D - D,), f32)])
    prm = jnp.stack([jnp.tile(b1, K), jnp.tile(bn1_g, K), jnp.tile(bn1_b, K),
                     jnp.tile(b2, K), jnp.tile(bn2_g, K), jnp.tile(bn2_b, K),
                     b3p, jnp.zeros((K * D,), f32)], axis=0)

    return _run_mlp(gl, xe, w1bd, w2bd, w3cat, prm)


# speedup vs baseline: 9.7919x; 1.1273x over previous
import functools

import jax
import jax.numpy as jnp
import numpy as np
from jax import lax
from jax.experimental import pallas as pl
from jax.experimental.pallas import tpu as pltpu
from jax.experimental.pallas import tpu_sc as plsc

N = 8192
M = 8192
D = 16
K = 16

BX = 256
CY = 1024
NX = N // BX
NCY = M // CY

_INF = 1e30
_BIGI = 1e9


def _topk_body(xblo, xbhi, yblo, ybhi, xa_ref, yt_ref, xbf_ref, ybf_ref,
               oidx_ref, topv, topi, abuf, ibuf):
    i = pl.program_id(0)
    j = pl.program_id(1)

    @pl.when(j == 0)
    def _init():
        topv[...] = jnp.full((BX, K), _INF, dtype=jnp.float32)
        topi[...] = jnp.zeros((BX, K), dtype=jnp.float32)

    overlap = (xblo[i] <= ybhi[j]) & (yblo[j] <= xbhi[i])

    @pl.when(overlap)
    def _merge():
        xa = xa_ref[...]
        yt = yt_ref[...]
        dot = lax.dot_general(xa, yt, (((1,), (0,)), ((), ())),
                              preferred_element_type=jnp.float32)
        sx = jnp.sum(xa * xa, axis=1, keepdims=True)
        sy = jnp.sum(yt * yt, axis=0, keepdims=True)
        d2 = (sx + sy) - 2.0 * dot
        pen = jnp.where(xbf_ref[...] != ybf_ref[...], 1e10, 0.0).astype(jnp.float32)
        abuf[...] = d2 + pen

        colbase = (j * CY).astype(jnp.float32)
        ibuf[...] = (lax.broadcasted_iota(jnp.int32, (BX, CY), 1)
                     .astype(jnp.float32) + colbase)

        lane = lax.broadcasted_iota(jnp.int32, (BX, K), 1)

        def rnd(r, c):
            ev, ei = c
            a = abuf[...]
            sv = topv[...]
            mm = jnp.minimum(jnp.min(a, axis=1, keepdims=True),
                             jnp.min(sv, axis=1, keepdims=True))
            sel_a = a <= mm
            sel_s = sv <= mm
            ii = jnp.minimum(
                jnp.min(jnp.where(sel_a, ibuf[...], _BIGI), axis=1, keepdims=True),
                jnp.min(jnp.where(sel_s, topi[...], _BIGI), axis=1, keepdims=True))
            abuf[...] = jnp.where(sel_a, _INF, a)
            topv[...] = jnp.where(sel_s, _INF, sv)
            ev = jnp.where(lane == r, mm, ev)
            ei = jnp.where(lane == r, ii, ei)
            return (ev, ei)

        ev, ei = lax.fori_loop(
            0, K, rnd,
            (jnp.zeros((BX, K), jnp.float32), jnp.zeros((BX, K), jnp.float32)))
        topv[...] = ev
        topi[...] = ei

    @pl.when(j == NCY - 1)
    def _out():
        oidx_ref[...] = topi[...].astype(jnp.int32)


def _run_topk(xa, yt, xbf, ybf, xblo, xbhi, yblo, ybhi):
    grid_spec = pltpu.PrefetchScalarGridSpec(
        num_scalar_prefetch=4,
        grid=(NX, NCY),
        in_specs=[
            pl.BlockSpec((BX, 8), lambda i, j, *_: (i, 0)),
            pl.BlockSpec((8, CY), lambda i, j, *_: (0, j)),
            pl.BlockSpec((BX, 1), lambda i, j, *_: (i, 0)),
            pl.BlockSpec((1, CY), lambda i, j, *_: (0, j)),
        ],
        out_specs=pl.BlockSpec((BX, K), lambda i, j, *_: (i, 0)),
        scratch_shapes=[
            pltpu.VMEM((BX, K), jnp.float32),
            pltpu.VMEM((BX, K), jnp.float32),
            pltpu.VMEM((BX, CY), jnp.float32),
            pltpu.VMEM((BX, CY), jnp.float32),
        ],
    )
    return pl.pallas_call(
        _topk_body,
        grid_spec=grid_spec,
        out_shape=jax.ShapeDtypeStruct((N, K), jnp.int32),
    )(xblo, xbhi, yblo, ybhi, xa, yt, xbf, ybf)


_NW = 32
_BPW = (N * K) // _NW
_HALF = _BPW // 2
_GCH = 128


def _sc_gather_body(tab_hbm, idx_hbm, out_hbm, idx_v, rows_v, sem):
    wid = lax.axis_index("s") * 2 + lax.axis_index("c")
    base = wid * _BPW
    pltpu.sync_copy(idx_hbm.at[pl.ds(base, _BPW)], idx_v)
    for h in range(2):
        hoff = h * _HALF
        copies = []
        for c in range(_HALF // _GCH):
            src = tab_hbm.at[idx_v.at[pl.ds(hoff + c * _GCH, _GCH)]]
            dst = rows_v.at[pl.ds(c * _GCH, _GCH)]
            copies.append(pltpu.async_copy(src, dst, sem))
        for cp in copies:
            cp.wait()
        pltpu.sync_copy(rows_v, out_hbm.at[pl.ds(base + hoff, _HALF)])


def _run_gather(tab, idxf):
    mesh = plsc.VectorSubcoreMesh(core_axis_name="c", subcore_axis_name="s")
    kfn = functools.partial(
        pl.kernel,
        mesh=mesh,
        compiler_params=pltpu.CompilerParams(use_tc_tiling_on_sc=False),
        out_type=jax.ShapeDtypeStruct((N * K, 32), jnp.float32),
        scratch_types=[
            pltpu.VMEM((_BPW,), jnp.int32),
            pltpu.VMEM((_HALF, 32), jnp.float32),
            pltpu.SemaphoreType.DMA,
        ],
    )(_sc_gather_body)
    return kfn(tab, idxf)


_KW = K * 32
_KD = K * D


def _mlp_body(g_ref, xe_ref, w1_ref, w2_ref, w3_ref, prm_ref, out_ref):
    gl = g_ref[...]
    xe = xe_ref[...]
    xew = jnp.concatenate([xe] * K, axis=1)

    lane = lax.broadcasted_iota(jnp.int32, (1, _KW), 1)
    lm = lax.rem(lane, 32)
    cmaskl = ((lm >= D) & (lm < D + 3)).astype(jnp.float32)
    amaskl = (lm < D).astype(jnp.float32)
    dhotl = lm == D

    diff = (gl - xew) * cmaskl
    sq = diff * diff
    t = (sq + jnp.roll(sq, -1, axis=1)) + jnp.roll(sq, -2, axis=1)
    rec = 1.0 / t
    featl = gl * amaskl + jnp.where(dhotl, rec, 0.0)

    s = float(np.sqrt(1.0 + 1e-5))
    b1 = prm_ref[0:1, :]
    g1 = prm_ref[1:2, :]
    be1 = prm_ref[2:3, :]
    b2 = prm_ref[3:4, :]
    g2 = prm_ref[4:5, :]
    be2 = prm_ref[5:6, :]
    b3 = prm_ref[6:7, 0:D]

    fx = lax.dot_general(featl, w1_ref[...], (((1,), (0,)), ((), ())),
                         preferred_element_type=jnp.float32) + b1
    fx = jnp.where(fx >= 0, fx, 0.2 * fx)
    fx = fx / s * g1 + be1

    fy = lax.dot_general(fx, w2_ref[...], (((1,), (0,)), ((), ())),
                         preferred_element_type=jnp.float32) + b2
    fy = jnp.where(fy >= 0, fy, 0.2 * fy)
    fy = fy / s * g2 + be2

    cc = jnp.concatenate([fx, fy], axis=1)
    out_ref[...] = lax.dot_general(cc, w3_ref[...], (((1,), (0,)), ((), ())),
                                   preferred_element_type=jnp.float32) + b3


def _run_mlp(g3, xe, w1p, w2p, w3p, prm):
    return pl.pallas_call(
        _mlp_body,
        grid=(NX,),
        in_specs=[
            pl.BlockSpec((BX, _KW), lambda i: (i, 0)),
            pl.BlockSpec((BX, 32), lambda i: (i, 0)),
            pl.BlockSpec((_KW, _KD), lambda i: (0, 0)),
            pl.BlockSpec((_KD, _KD), lambda i: (0, 0)),
            pl.BlockSpec((2 * _KD, D), lambda i: (0, 0)),
            pl.BlockSpec((8, _KD), lambda i: (0, 0)),
        ],
        out_specs=pl.BlockSpec((BX, D), lambda i: (i, 0)),
        out_shape=jax.ShapeDtypeStruct((N, D), jnp.float32),
    )(g3, xe, w1p, w2p, w3p, prm)


def kernel(x, y, y_atomtypes, W1, b1, W2, b2, W3, b3,
           bn1_g, bn1_b, bn2_g, bn2_b, x_batch, y_batch):
    f32 = jnp.float32
    x = x.astype(f32)
    y = y.astype(f32)

    xa = jnp.concatenate([x, jnp.zeros((N, 5), f32)], axis=1)
    yt = jnp.concatenate([y, jnp.zeros((M, 5), f32)], axis=1).T
    xbf = x_batch.astype(f32)[:, None]
    ybf = y_batch.astype(f32)[None, :]
    xb2 = x_batch.reshape(NX, BX).astype(jnp.int32)
    yb2 = y_batch.reshape(NCY, CY).astype(jnp.int32)
    xblo, xbhi = xb2.min(axis=1), xb2.max(axis=1)
    yblo, ybhi = yb2.min(axis=1), yb2.max(axis=1)

    idx = _run_topk(xa, yt, xbf, ybf, xblo, xbhi, yblo, ybhi)

    tab = jnp.concatenate([y_atomtypes.astype(f32), y,
                           jnp.zeros((M, 13), f32)], axis=1)
    gath = _run_gather(tab, idx.reshape(N * K))

    gl = gath.reshape(N, K * 32)
    xe = jnp.concatenate([jnp.zeros((N, D), f32), x,
                          jnp.zeros((N, 32 - D - 3), f32)], axis=1)
    w1p = jnp.concatenate([W1[:, :D].T, W1[:, D:D + 1].T,
                           jnp.zeros((32 - D - 1, D), f32)], axis=0)
    eyek = jnp.eye(K, dtype=f32)
    w1bd = jnp.kron(eyek, w1p)
    w2bd = jnp.kron(eyek, W2.T)
    w3cat = jnp.concatenate([jnp.tile(W3.T[:D], (K, 1)),
                             jnp.tile(W3.T[D:], (K, 1))], axis=0)
    b3p = jnp.concatenate([b3, jnp.zeros((K * D - D,), f32)])
    prm = jnp.stack([jnp.tile(b1, K), jnp.tile(bn1_g, K), jnp.tile(bn1_b, K),
                     jnp.tile(b2, K), jnp.tile(bn2_g, K), jnp.tile(bn2_b, K),
                     b3p, jnp.zeros((K * D,), f32)], axis=0)

    return _run_mlp(gl, xe, w1bd, w2bd, w3cat, prm)


# speedup vs baseline: 16.5335x; 1.6885x over previous
import functools

import jax
import jax.numpy as jnp
import numpy as np
from jax import lax
from jax.experimental import pallas as pl
from jax.experimental.pallas import tpu as pltpu
from jax.experimental.pallas import tpu_sc as plsc

N = 8192
M = 8192
D = 16
K = 16

BX = 512
CY = 1024
NX = N // BX
NCY = M // CY

_INF = 1e30
_BIGI = 1e9


HCY = CY // 2


def _topk_body(xblo, xbhi, yblo, ybhi, xa_ref, yt_ref, xbf_ref, ybf_ref,
               oidx_ref, pbuf, ipbuf, qbuf, iqbuf):
    i = pl.program_id(0)
    j = pl.program_id(1)

    @pl.when(j == 0)
    def _init():
        pbuf[:, HCY:] = jnp.full((BX, K), _INF, dtype=jnp.float32)
        ipbuf[:, HCY:] = jnp.zeros((BX, K), dtype=jnp.float32)

    overlap = (xblo[i] <= ybhi[j]) & (yblo[j] <= xbhi[i])

    @pl.when(overlap)
    def _merge():
        xa = xa_ref[...]
        yt = yt_ref[...]
        dot = lax.dot_general(xa, yt, (((1,), (0,)), ((), ())),
                              preferred_element_type=jnp.float32)
        sx = jnp.sum(xa * xa, axis=1, keepdims=True)
        sy = jnp.sum(yt * yt, axis=0, keepdims=True)
        d2 = (sx + sy) - 2.0 * dot
        pen = jnp.where(xbf_ref[...] != ybf_ref[...], 1e10, 0.0).astype(jnp.float32)
        d2p = d2 + pen

        colbase = (j * CY).astype(jnp.float32)
        il = (lax.broadcasted_iota(jnp.int32, (BX, HCY), 1)
              .astype(jnp.float32) + colbase)
        lo = d2p[:, :HCY]
        hi = d2p[:, HCY:]
        selp = lo <= hi
        pbuf[:, :HCY] = jnp.where(selp, lo, hi)
        qbuf[...] = jnp.where(selp, hi, lo)
        ipbuf[:, :HCY] = jnp.where(selp, il, il + HCY)
        iqbuf[...] = jnp.where(selp, il + HCY, il)

        lane = lax.broadcasted_iota(jnp.int32, (BX, K), 1)

        def rnd(r, c):
            ev, ei = c
            p = pbuf[...]
            mm = jnp.min(p, axis=1, keepdims=True)
            sel = p <= mm
            ip = ipbuf[...]
            ii = jnp.min(jnp.where(sel, ip, _BIGI), axis=1, keepdims=True)
            q = qbuf[...]
            sel_c = sel[:, :HCY]
            pbuf[:, :HCY] = jnp.where(sel_c, q, p[:, :HCY])
            pbuf[:, HCY:] = jnp.where(sel[:, HCY:], _INF, p[:, HCY:])
            ipbuf[:, :HCY] = jnp.where(sel_c, iqbuf[...], ip[:, :HCY])
            qbuf[...] = jnp.where(sel_c, _INF, q)
            ev = jnp.where(lane == r, mm, ev)
            ei = jnp.where(lane == r, ii, ei)
            return (ev, ei)

        ev, ei = lax.fori_loop(
            0, K, rnd,
            (jnp.zeros((BX, K), jnp.float32), jnp.zeros((BX, K), jnp.float32)),
            unroll=16)
        pbuf[:, HCY:] = ev
        ipbuf[:, HCY:] = ei

    @pl.when(j == NCY - 1)
    def _out():
        oidx_ref[...] = ipbuf[:, HCY:].astype(jnp.int32)


def _run_topk(xa, yt, xbf, ybf, xblo, xbhi, yblo, ybhi):
    grid_spec = pltpu.PrefetchScalarGridSpec(
        num_scalar_prefetch=4,
        grid=(NX, NCY),
        in_specs=[
            pl.BlockSpec((BX, 8), lambda i, j, *_: (i, 0)),
            pl.BlockSpec((8, CY), lambda i, j, *_: (0, j)),
            pl.BlockSpec((BX, 1), lambda i, j, *_: (i, 0)),
            pl.BlockSpec((1, CY), lambda i, j, *_: (0, j)),
        ],
        out_specs=pl.BlockSpec((BX, K), lambda i, j, *_: (i, 0)),
        scratch_shapes=[
            pltpu.VMEM((BX, HCY + K), jnp.float32),
            pltpu.VMEM((BX, HCY + K), jnp.float32),
            pltpu.VMEM((BX, HCY), jnp.float32),
            pltpu.VMEM((BX, HCY), jnp.float32),
        ],
    )
    return pl.pallas_call(
        _topk_body,
        grid_spec=grid_spec,
        out_shape=jax.ShapeDtypeStruct((N, K), jnp.int32),
    )(xblo, xbhi, yblo, ybhi, xa, yt, xbf, ybf)


_NW = 32
_BPW = (N * K) // _NW
_HALF = _BPW // 2
_GCH = 128


def _sc_gather_body(tab_hbm, idx_hbm, out_hbm, idx_v, rows_v, sem):
    wid = lax.axis_index("s") * 2 + lax.axis_index("c")
    base = wid * _BPW
    pltpu.sync_copy(idx_hbm.at[pl.ds(base, _BPW)], idx_v)
    for h in range(2):
        hoff = h * _HALF
        copies = []
        for c in range(_HALF // _GCH):
            src = tab_hbm.at[idx_v.at[pl.ds(hoff + c * _GCH, _GCH)]]
            dst = rows_v.at[pl.ds(c * _GCH, _GCH)]
            copies.append(pltpu.async_copy(src, dst, sem))
        for cp in copies:
            cp.wait()
        pltpu.sync_copy(rows_v, out_hbm.at[pl.ds(base + hoff, _HALF)])


def _run_gather(tab, idxf):
    mesh = plsc.VectorSubcoreMesh(core_axis_name="c", subcore_axis_name="s")
    kfn = functools.partial(
        pl.kernel,
        mesh=mesh,
        compiler_params=pltpu.CompilerParams(use_tc_tiling_on_sc=False),
        out_type=jax.ShapeDtypeStruct((N * K, 32), jnp.float32),
        scratch_types=[
            pltpu.VMEM((_BPW,), jnp.int32),
            pltpu.VMEM((_HALF, 32), jnp.float32),
            pltpu.SemaphoreType.DMA,
        ],
    )(_sc_gather_body)
    return kfn(tab, idxf)


_KW = K * 32
_KD = K * D


def _mlp_body(g_ref, xe_ref, w1_ref, w2_ref, w3_ref, prm_ref, out_ref):
    gl = g_ref[...]
    xe = xe_ref[...]
    xew = jnp.concatenate([xe] * K, axis=1)

    lane = lax.broadcasted_iota(jnp.int32, (1, _KW), 1)
    lm = lax.rem(lane, 32)
    cmaskl = ((lm >= D) & (lm < D + 3)).astype(jnp.float32)
    amaskl = (lm < D).astype(jnp.float32)
    dhotl = lm == D

    diff = (gl - xew) * cmaskl
    sq = diff * diff
    t = (sq + jnp.roll(sq, -1, axis=1)) + jnp.roll(sq, -2, axis=1)
    rec = 1.0 / t
    featl = gl * amaskl + jnp.where(dhotl, rec, 0.0)

    s = float(np.sqrt(1.0 + 1e-5))
    b1 = prm_ref[0:1, :]
    g1 = prm_ref[1:2, :]
    be1 = prm_ref[2:3, :]
    b2 = prm_ref[3:4, :]
    g2 = prm_ref[4:5, :]
    be2 = prm_ref[5:6, :]
    b3 = prm_ref[6:7, 0:D]

    fx = lax.dot_general(featl, w1_ref[...], (((1,), (0,)), ((), ())),
                         preferred_element_type=jnp.float32) + b1
    fx = jnp.where(fx >= 0, fx, 0.2 * fx)
    fx = fx / s * g1 + be1

    fy = lax.dot_general(fx, w2_ref[...], (((1,), (0,)), ((), ())),
                         preferred_element_type=jnp.float32) + b2
    fy = jnp.where(fy >= 0, fy, 0.2 * fy)
    fy = fy / s * g2 + be2

    cc = jnp.concatenate([fx, fy], axis=1)
    out_ref[...] = lax.dot_general(cc, w3_ref[...], (((1,), (0,)), ((), ())),
                                   preferred_element_type=jnp.float32) + b3


def _run_mlp(g3, xe, w1p, w2p, w3p, prm):
    return pl.pallas_call(
        _mlp_body,
        grid=(NX,),
        in_specs=[
            pl.BlockSpec((BX, _KW), lambda i: (i, 0)),
            pl.BlockSpec((BX, 32), lambda i: (i, 0)),
            pl.BlockSpec((_KW, _KD), lambda i: (0, 0)),
            pl.BlockSpec((_KD, _KD), lambda i: (0, 0)),
            pl.BlockSpec((2 * _KD, D), lambda i: (0, 0)),
            pl.BlockSpec((8, _KD), lambda i: (0, 0)),
        ],
        out_specs=pl.BlockSpec((BX, D), lambda i: (i, 0)),
        out_shape=jax.ShapeDtypeStruct((N, D), jnp.float32),
    )(g3, xe, w1p, w2p, w3p, prm)


def kernel(x, y, y_atomtypes, W1, b1, W2, b2, W3, b3,
           bn1_g, bn1_b, bn2_g, bn2_b, x_batch, y_batch):
    f32 = jnp.float32
    x = x.astype(f32)
    y = y.astype(f32)

    xa = jnp.concatenate([x, jnp.zeros((N, 5), f32)], axis=1)
    yt = jnp.concatenate([y, jnp.zeros((M, 5), f32)], axis=1).T
    xbf = x_batch.astype(f32)[:, None]
    ybf = y_batch.astype(f32)[None, :]
    xb2 = x_batch.reshape(NX, BX).astype(jnp.int32)
    yb2 = y_batch.reshape(NCY, CY).astype(jnp.int32)
    xblo, xbhi = xb2.min(axis=1), xb2.max(axis=1)
    yblo, ybhi = yb2.min(axis=1), yb2.max(axis=1)

    idx = _run_topk(xa, yt, xbf, ybf, xblo, xbhi, yblo, ybhi)

    tab = jnp.concatenate([y_atomtypes.astype(f32), y,
                           jnp.zeros((M, 13), f32)], axis=1)
    gath = _run_gather(tab, idx.reshape(N * K))

    gl = gath.reshape(N, K * 32)
    xe = jnp.concatenate([jnp.zeros((N, D), f32), x,
                          jnp.zeros((N, 32 - D - 3), f32)], axis=1)
    w1p = jnp.concatenate([W1[:, :D].T, W1[:, D:D + 1].T,
                           jnp.zeros((32 - D - 1, D), f32)], axis=0)
    eyek = jnp.eye(K, dtype=f32)
    w1bd = jnp.kron(eyek, w1p)
    w2bd = jnp.kron(eyek, W2.T)
    w3cat = jnp.concatenate([jnp.tile(W3.T[:D], (K, 1)),
                             jnp.tile(W3.T[D:], (K, 1))], axis=0)
    b3p = jnp.concatenate([b3, jnp.zeros((K * D - D,), f32)])
    prm = jnp.stack([jnp.tile(b1, K), jnp.tile(bn1_g, K), jnp.tile(bn1_b, K),
                     jnp.tile(b2, K), jnp.tile(bn2_g, K), jnp.tile(bn2_b, K),
                     b3p, jnp.zeros((K * D,), f32)], axis=0)

    return _run_mlp(gl, xe, w1bd, w2bd, w3cat, prm)
